# baseline (speedup 1.0000x reference)
import functools

import jax
import jax.numpy as jnp
from jax import lax
from jax.experimental import pallas as pl
from jax.experimental.pallas import tpu as pltpu
from jax.experimental.pallas import tpu_sc as plsc

N = 10000
NPAD = 10240
D = 256
H = 128
E = 160000
EPAD = 163840
NC = 2
NS = 16
CHUNK = 128
ECHUNKS_TILE = EPAD // (NS * CHUNK)
DCHUNKS_TILE = EPAD // (NC * NS * CHUNK)
STRIPE = NPAD // NS
ROWBLK = 400
NRB = N // ROWBLK
NBUF = 2

_mesh = plsc.VectorSubcoreMesh(core_axis_name="c", subcore_axis_name="s")


@functools.partial(
    pl.kernel,
    out_type=jax.ShapeDtypeStruct((NC, NPAD + 8), jnp.float32),
    mesh=_mesh,
    scratch_types=[
        pltpu.VMEM((DCHUNKS_TILE, CHUNK), jnp.int32),
        pltpu.VMEM((CHUNK,), jnp.float32),
        pltpu.VMEM((STRIPE,), jnp.float32),
        pltpu.VMEM_SHARED((NPAD + 8,), jnp.float32),
    ],
)
def _deg_kernel(dst_hbm, out_hbm, di, ones, zbuf, hist):
    c = lax.axis_index("c")
    s = lax.axis_index("s")
    tid = c * NS + s
    for j in range(CHUNK // 16):
        ones[pl.ds(16 * j, 16)] = jnp.ones((16,), jnp.float32)
    for j in range(STRIPE // 16):
        zbuf[pl.ds(16 * j, 16)] = jnp.zeros((16,), jnp.float32)
    pltpu.sync_copy(zbuf, hist.at[pl.ds(s * STRIPE, STRIPE)])
    pltpu.sync_copy(dst_hbm.at[tid], di)
    plsc.subcore_barrier()

    @pl.loop(0, DCHUNKS_TILE)
    def _(g):
        pltpu.sync_copy(ones, hist.at[di.at[g]], add=True)

    plsc.subcore_barrier()
    pltpu.sync_copy(hist.at[pl.ds(s * STRIPE, STRIPE)],
                    out_hbm.at[c, pl.ds(s * STRIPE, STRIPE)])


NSI = 4


@functools.partial(
    pl.kernel,
    out_type=jax.ShapeDtypeStruct((NC, NPAD, H), jnp.float32),
    mesh=_mesh,
    scratch_types=[
        pltpu.VMEM((ECHUNKS_TILE, CHUNK), jnp.int32),
        pltpu.VMEM((NSI, CHUNK), jnp.int32),
        pltpu.VMEM((CHUNK, H), jnp.float32),
        pltpu.VMEM((CHUNK, H), jnp.float32),
        pltpu.VMEM_SHARED((NPAD + 8, H), jnp.float32),
    ] + [pltpu.SemaphoreType.DMA] * (2 * NBUF + NSI),
)
def _edge_kernel(src_hbm, dst_hbm, y_hbm, z_hbm,
                 si, di, rows0, rows1, z, *sems):
    sg, ss, sd = sems[:NBUF], sems[NBUF:2 * NBUF], sems[2 * NBUF:]
    rows = (rows0, rows1)
    c = lax.axis_index("c")
    s = lax.axis_index("s")
    tid = c * NS + s

    @pl.loop(0, CHUNK)
    def _(r):
        for j in range(H // 16):
            rows0[r, pl.ds(16 * j, 16)] = jnp.zeros((16,), jnp.float32)

    for q in range(STRIPE // CHUNK):
        pltpu.sync_copy(rows0, z.at[pl.ds(s * STRIPE + q * CHUNK, CHUNK), :])

    pltpu.sync_copy(src_hbm.at[tid], si)
    plsc.subcore_barrier()

    def load_di(k, slot):
        pltpu.async_copy(dst_hbm.at[tid * ECHUNKS_TILE + k], di.at[slot], sd[slot])

    for j in range(NSI):
        load_di(j, j)
    for b in range(NBUF):
        pltpu.async_copy(y_hbm.at[si.at[b]], rows[b], sg[b])

    @pl.loop(0, ECHUNKS_TILE, step=NSI)
    def _(g):
        for b in range(NSI):
            k = g + b
            rb = b % NBUF
            pltpu.make_async_copy(y_hbm.at[si.at[k]], rows[rb],
                                  sg[rb]).wait()
            pltpu.make_async_copy(
                dst_hbm.at[tid * ECHUNKS_TILE + k], di.at[b], sd[b]).wait()
            pltpu.async_copy(rows[rb], z.at[di.at[b]], ss[rb], add=True)

            @pl.when(k + NBUF < ECHUNKS_TILE)
            def _():
                pltpu.make_async_copy(rows[rb], z.at[di.at[b]],
                                      ss[rb]).wait()
                pltpu.async_copy(y_hbm.at[si.at[k + NBUF]], rows[rb], sg[rb])

                @pl.when(k + NSI < ECHUNKS_TILE)
                def _():
                    load_di(k + NSI, b)

    for b in range(NBUF):
        e = ECHUNKS_TILE - NBUF + b
        pltpu.make_async_copy(rows[e % NBUF], z.at[di.at[e % NSI]],
                              ss[e % NBUF]).wait()

    plsc.subcore_barrier()
    pltpu.sync_copy(z.at[pl.ds(s * STRIPE, STRIPE), :],
                    z_hbm.at[c, pl.ds(s * STRIPE, STRIPE), :])


def _mm_body(x_ref, w_ref, deg_ref, y_ref):
    xw = jnp.dot(x_ref[...], w_ref[...], preferred_element_type=jnp.float32)
    dinv = lax.rsqrt(deg_ref[:, 0] + deg_ref[:, 1] + 1.0)
    y_ref[...] = xw * dinv[:, None]


def _matmul(x, w, degp):
    return pl.pallas_call(
        _mm_body,
        grid=(NC, NRB),
        in_specs=[
            pl.BlockSpec((ROWBLK, D), lambda h, i: (i, 0)),
            pl.BlockSpec((D, H), lambda h, i: (0, h)),
            pl.BlockSpec((ROWBLK, NC), lambda h, i: (i, 0)),
        ],
        out_specs=pl.BlockSpec((ROWBLK, H), lambda h, i: (h * NRB + i, 0)),
        out_shape=jax.ShapeDtypeStruct((NC * N, H), jnp.float32),
    )(x, w, degp)


def _fin_body(z0_ref, z1_ref, y0_ref, y1_ref, deg_ref, b_ref, out_ref):
    dinv = lax.rsqrt(deg_ref[:, 0] + deg_ref[:, 1] + 1.0)[:, None]
    b = b_ref[0, :]
    oL = (z0_ref[0] + y0_ref[...]) * dinv + b[None, :H]
    oR = (z1_ref[0] + y1_ref[...]) * dinv + b[None, H:]
    o = jnp.concatenate([oL, oR], axis=1)
    m = jnp.max(o, axis=1, keepdims=True)
    lse = jnp.log(jnp.sum(jnp.exp(o - m), axis=1, keepdims=True)) + m
    out_ref[...] = o - lse


def _final(z_all, y_all, degp, b2):
    return pl.pallas_call(
        _fin_body,
        grid=(NRB,),
        in_specs=[
            pl.BlockSpec((1, ROWBLK, H), lambda i: (0, i, 0)),
            pl.BlockSpec((1, ROWBLK, H), lambda i: (1, i, 0)),
            pl.BlockSpec((ROWBLK, H), lambda i: (i, 0)),
            pl.BlockSpec((ROWBLK, H), lambda i: (NRB + i, 0)),
            pl.BlockSpec((ROWBLK, NC), lambda i: (i, 0)),
            pl.BlockSpec((1, D), lambda i: (0, 0)),
        ],
        out_specs=pl.BlockSpec((ROWBLK, D), lambda i: (i, 0)),
        out_shape=jax.ShapeDtypeStruct((N, D), jnp.float32),
    )(z_all, z_all, y_all, y_all, degp, b2)


def kernel(x, edge_index, W, b):
    ei = edge_index.astype(jnp.int32)
    npad_e = EPAD - E
    src = jnp.concatenate([ei[0], jnp.zeros((npad_e,), jnp.int32)])
    dst = jnp.concatenate([ei[1], jnp.full((npad_e,), NPAD, jnp.int32)])
    src3 = src.reshape(NS, ECHUNKS_TILE, CHUNK)
    dst3 = dst.reshape(NS, ECHUNKS_TILE, CHUNK)
    si_all = jnp.stack([src3, src3 + N])
    dst_edge = jnp.broadcast_to(dst3, (NC, NS, ECHUNKS_TILE, CHUNK)).reshape(
        NC * NS * ECHUNKS_TILE, CHUNK)
    dst_deg = dst.reshape(NC * NS, DCHUNKS_TILE, CHUNK)

    degp = _deg_kernel(dst_deg)[:, :NPAD].T
    y_all = _matmul(x, W, degp)
    z_all = _edge_kernel(si_all.reshape(NC * NS, ECHUNKS_TILE, CHUNK), dst_edge, y_all)
    return _final(z_all, y_all, degp, b.reshape(1, D))


# speedup vs baseline: 1.0172x; 1.0172x over previous
import functools

import jax
import jax.numpy as jnp
from jax import lax
from jax.experimental import pallas as pl
from jax.experimental.pallas import tpu as pltpu
from jax.experimental.pallas import tpu_sc as plsc

N = 10000
NPAD = 10240
D = 256
H = 128
E = 160000
EPAD = 163840
NC = 2
NS = 16
CHUNK = 128
ECHUNKS_TILE = EPAD // (NS * CHUNK)
DCHUNKS_TILE = EPAD // (NC * NS * CHUNK)
STRIPE = NPAD // NS
ROWBLK = 512
NRB = NPAD // ROWBLK
NBUF = 2

_mesh = plsc.VectorSubcoreMesh(core_axis_name="c", subcore_axis_name="s")


@functools.partial(
    pl.kernel,
    out_type=jax.ShapeDtypeStruct((NC, NPAD + 8), jnp.float32),
    mesh=_mesh,
    scratch_types=[
        pltpu.VMEM((DCHUNKS_TILE, CHUNK), jnp.int32),
        pltpu.VMEM((CHUNK,), jnp.float32),
        pltpu.VMEM((STRIPE,), jnp.float32),
        pltpu.VMEM_SHARED((NPAD + 8,), jnp.float32),
    ],
)
def _deg_kernel(dst_hbm, out_hbm, di, ones, zbuf, hist):
    c = lax.axis_index("c")
    s = lax.axis_index("s")
    tid = c * NS + s
    for j in range(CHUNK // 16):
        ones[pl.ds(16 * j, 16)] = jnp.ones((16,), jnp.float32)
    for j in range(STRIPE // 16):
        zbuf[pl.ds(16 * j, 16)] = jnp.zeros((16,), jnp.float32)
    pltpu.sync_copy(zbuf, hist.at[pl.ds(s * STRIPE, STRIPE)])
    pltpu.sync_copy(dst_hbm.at[tid], di)
    plsc.subcore_barrier()

    @pl.loop(0, DCHUNKS_TILE)
    def _(g):
        pltpu.sync_copy(ones, hist.at[di.at[g]], add=True)

    plsc.subcore_barrier()
    pltpu.sync_copy(hist.at[pl.ds(s * STRIPE, STRIPE)],
                    out_hbm.at[c, pl.ds(s * STRIPE, STRIPE)])


NSI = 4


@functools.partial(
    pl.kernel,
    out_type=jax.ShapeDtypeStruct((NC * NPAD, H), jnp.float32),
    mesh=_mesh,
    scratch_types=[
        pltpu.VMEM((ECHUNKS_TILE, CHUNK), jnp.int32),
        pltpu.VMEM((NSI, CHUNK), jnp.int32),
        pltpu.VMEM((CHUNK, H), jnp.float32),
        pltpu.VMEM((CHUNK, H), jnp.float32),
        pltpu.VMEM_SHARED((NPAD + 8, H), jnp.float32),
    ] + [pltpu.SemaphoreType.DMA] * (2 * NBUF + NSI),
)
def _edge_kernel(src_hbm, dst_hbm, y_hbm, z_hbm,
                 si, di, rows0, rows1, z, *sems):
    sg, ss, sd = sems[:NBUF], sems[NBUF:2 * NBUF], sems[2 * NBUF:]
    rows = (rows0, rows1)
    c = lax.axis_index("c")
    s = lax.axis_index("s")
    tid = c * NS + s

    @pl.loop(0, CHUNK)
    def _(r):
        for j in range(H // 16):
            rows0[r, pl.ds(16 * j, 16)] = jnp.zeros((16,), jnp.float32)

    for q in range(STRIPE // CHUNK):
        pltpu.sync_copy(rows0, z.at[pl.ds(s * STRIPE + q * CHUNK, CHUNK), :])

    pltpu.sync_copy(src_hbm.at[tid], si)
    plsc.subcore_barrier()

    def load_di(k, slot):
        pltpu.async_copy(dst_hbm.at[tid * ECHUNKS_TILE + k], di.at[slot], sd[slot])

    for j in range(NSI):
        load_di(j, j)
    for b in range(NBUF):
        pltpu.async_copy(y_hbm.at[si.at[b]], rows[b], sg[b])

    @pl.loop(0, ECHUNKS_TILE, step=NSI)
    def _(g):
        for b in range(NSI):
            k = g + b
            rb = b % NBUF
            pltpu.make_async_copy(y_hbm.at[si.at[k]], rows[rb],
                                  sg[rb]).wait()
            pltpu.make_async_copy(
                dst_hbm.at[tid * ECHUNKS_TILE + k], di.at[b], sd[b]).wait()
            pltpu.async_copy(rows[rb], z.at[di.at[b]], ss[rb], add=True)

            @pl.when(k + NBUF < ECHUNKS_TILE)
            def _():
                pltpu.make_async_copy(rows[rb], z.at[di.at[b]],
                                      ss[rb]).wait()
                pltpu.async_copy(y_hbm.at[si.at[k + NBUF]], rows[rb], sg[rb])

                @pl.when(k + NSI < ECHUNKS_TILE)
                def _():
                    load_di(k + NSI, b)

    for b in range(NBUF):
        e = ECHUNKS_TILE - NBUF + b
        pltpu.make_async_copy(rows[e % NBUF], z.at[di.at[e % NSI]],
                              ss[e % NBUF]).wait()

    plsc.subcore_barrier()
    pltpu.sync_copy(z.at[pl.ds(s * STRIPE, STRIPE), :],
                    z_hbm.at[pl.ds(c * NPAD + s * STRIPE, STRIPE), :])


def _mm_body(x_ref, w_ref, deg_ref, y_ref):
    xw = jnp.dot(x_ref[...], w_ref[...], preferred_element_type=jnp.float32)
    dinv = lax.rsqrt(deg_ref[0, :] + deg_ref[1, :] + 1.0)
    y_ref[...] = xw * dinv[:, None]


def _matmul(xpad, w, degp):
    return pl.pallas_call(
        _mm_body,
        grid=(NC, NRB),
        in_specs=[
            pl.BlockSpec((ROWBLK, D), lambda h, i: (i, 0)),
            pl.BlockSpec((D, H), lambda h, i: (0, h)),
            pl.BlockSpec((NC, ROWBLK), lambda h, i: (0, i)),
        ],
        out_specs=pl.BlockSpec((ROWBLK, H), lambda h, i: (h * NRB + i, 0)),
        out_shape=jax.ShapeDtypeStruct((NC * NPAD, H), jnp.float32),
    )(xpad, w, degp)


def _fin_body(z0_ref, z1_ref, y0_ref, y1_ref, deg_ref, b_ref, out_ref):
    dinv = lax.rsqrt(deg_ref[0, :] + deg_ref[1, :] + 1.0)[:, None]
    b = b_ref[0, :]
    oL = (z0_ref[...] + y0_ref[...]) * dinv + b[None, :H]
    oR = (z1_ref[...] + y1_ref[...]) * dinv + b[None, H:]
    o = jnp.concatenate([oL, oR], axis=1)
    m = jnp.max(o, axis=1, keepdims=True)
    lse = jnp.log(jnp.sum(jnp.exp(o - m), axis=1, keepdims=True)) + m
    out_ref[...] = o - lse


def _final(z_all, y_all, degp, b2):
    lo = lambda i: (i, 0)
    hi = lambda i: (NRB + i, 0)
    return pl.pallas_call(
        _fin_body,
        grid=(NRB,),
        in_specs=[
            pl.BlockSpec((ROWBLK, H), lo),
            pl.BlockSpec((ROWBLK, H), hi),
            pl.BlockSpec((ROWBLK, H), lo),
            pl.BlockSpec((ROWBLK, H), hi),
            pl.BlockSpec((NC, ROWBLK), lambda i: (0, i)),
            pl.BlockSpec((1, D), lambda i: (0, 0)),
        ],
        out_specs=pl.BlockSpec((ROWBLK, D), lo),
        out_shape=jax.ShapeDtypeStruct((N, D), jnp.float32),
    )(z_all, z_all, y_all, y_all, degp, b2)


def kernel(x, edge_index, W, b):
    ei = edge_index.astype(jnp.int32)
    npad_e = EPAD - E
    src = jnp.concatenate([ei[0], jnp.zeros((npad_e,), jnp.int32)])
    dst = jnp.concatenate([ei[1], jnp.full((npad_e,), NPAD, jnp.int32)])
    src3 = src.reshape(NS, ECHUNKS_TILE, CHUNK)
    dst3 = dst.reshape(NS, ECHUNKS_TILE, CHUNK)
    si_all = jnp.stack([src3, src3 + NPAD])
    dst_edge = jnp.broadcast_to(dst3, (NC, NS, ECHUNKS_TILE, CHUNK)).reshape(
        NC * NS * ECHUNKS_TILE, CHUNK)
    dst_deg = dst.reshape(NC * NS, DCHUNKS_TILE, CHUNK)

    degp = _deg_kernel(dst_deg)[:, :NPAD]
    xpad = jnp.pad(x, ((0, NPAD - N), (0, 0)))
    y_all = _matmul(xpad, W, degp)
    z_all = _edge_kernel(si_all.reshape(NC * NS, ECHUNKS_TILE, CHUNK), dst_edge, y_all)
    return _final(z_all, y_all, degp, b.reshape(1, D))
